# Initial kernel scaffold; baseline (speedup 1.0000x reference)
import functools

import jax
import jax.numpy as jnp
from jax import lax
from jax.experimental import pallas as pl
from jax.experimental.pallas import tpu as pltpu
from jax.experimental.pallas import tpu_sc as plsc

B = 16384
L = 200
HID = 32
NLANE = 16

_INFO = plsc.get_sparse_core_info()
NC = _INFO.num_cores
NS = _INFO.num_subcores
NW = NC * NS
SPW = B // NW

G = 8
NG = SPW // G
IDS_PER_G = G * L
CHUNK = 80
NCHUNK = IDS_PER_G // CHUNK


def _sc_embed_body(table_hbm, ids_hbm, out_hbm, idx_v, rows_v, out_v, sem):
    wid = lax.axis_index("s") * NC + lax.axis_index("c")
    inv_l = jnp.float32(1.0 / L)

    def group_body(g, carry):
        ids_base = wid * (SPW * L) + g * IDS_PER_G
        pltpu.sync_copy(ids_hbm.at[pl.ds(ids_base, IDS_PER_G)], idx_v)

        copies = []
        for j in range(NCHUNK):
            copies.append(
                pltpu.async_copy(
                    table_hbm.at[idx_v.at[pl.ds(j * CHUNK, CHUNK)]],
                    rows_v.at[pl.ds(j * CHUNK, CHUNK)],
                    sem,
                )
            )
        for c in copies:
            c.wait()

        def sample_body(s, carry2):
            rb = s * L

            def add2(l, acc):
                a0, a1, b0, b1 = acc
                r = rb + 2 * l
                a0 = a0 + rows_v[r, pl.ds(0, NLANE)]
                a1 = a1 + rows_v[r, pl.ds(NLANE, NLANE)]
                b0 = b0 + rows_v[r + 1, pl.ds(0, NLANE)]
                b1 = b1 + rows_v[r + 1, pl.ds(NLANE, NLANE)]
                return a0, a1, b0, b1

            z = jnp.zeros((NLANE,), jnp.float32)
            a0, a1, b0, b1 = lax.fori_loop(
                0, L // 2, add2, (z, z, z, z), unroll=4
            )
            out_v[s, pl.ds(0, NLANE)] = (a0 + b0) * inv_l
            out_v[s, pl.ds(NLANE, NLANE)] = (a1 + b1) * inv_l
            return carry2

        lax.fori_loop(0, G, sample_body, 0)
        pltpu.sync_copy(out_v, out_hbm.at[pl.ds(wid * SPW + g * G, G)])
        return carry

    lax.fori_loop(0, NG, group_body, 0)


_sc_embed = functools.partial(
    pl.kernel,
    mesh=plsc.VectorSubcoreMesh(core_axis_name="c", subcore_axis_name="s"),
    out_type=jax.ShapeDtypeStruct((B, HID), jnp.float32),
    scratch_types=[
        pltpu.VMEM((IDS_PER_G,), jnp.int32),
        pltpu.VMEM((IDS_PER_G, HID), jnp.float32),
        pltpu.VMEM((G, HID), jnp.float32),
        pltpu.SemaphoreType.DMA,
    ],
)(_sc_embed_body)


def _head_body(emb_ref, w_ref, b_ref, lab_ref, logits_ref, loss_ref):
    emb = emb_ref[...]
    w = w_ref[...]
    l0 = jnp.sum(emb * w[0:1, :], axis=1, keepdims=True) + b_ref[0]
    l1 = jnp.sum(emb * w[1:2, :], axis=1, keepdims=True) + b_ref[1]
    logits_ref[...] = jnp.concatenate([l0, l1], axis=1)
    m = jnp.maximum(l0, l1)
    lse = m + jnp.log(jnp.exp(l0 - m) + jnp.exp(l1 - m))
    lab = lab_ref[...]
    lp = jnp.where(lab == 0, l0, l1) - lse
    loss_ref[0, 0] = -jnp.mean(lp)


_head = pl.pallas_call(
    _head_body,
    out_shape=(
        jax.ShapeDtypeStruct((B, 2), jnp.float32),
        jax.ShapeDtypeStruct((1, 1), jnp.float32),
    ),
    in_specs=[
        pl.BlockSpec(memory_space=pltpu.VMEM),
        pl.BlockSpec(memory_space=pltpu.VMEM),
        pl.BlockSpec(memory_space=pltpu.SMEM),
        pl.BlockSpec(memory_space=pltpu.VMEM),
    ],
    out_specs=(
        pl.BlockSpec(memory_space=pltpu.VMEM),
        pl.BlockSpec(memory_space=pltpu.SMEM),
    ),
)


def kernel(input_ids, labels, emb_table, W, b):
    ids_flat = input_ids.reshape(-1)
    emb = _sc_embed(emb_table, ids_flat)
    logits, loss = _head(emb, W, b, labels.reshape(B, 1))
    return loss[0, 0], logits


# speedup vs baseline: 13.2702x; 13.2702x over previous
import functools

import jax
import jax.numpy as jnp
from jax import lax
from jax.experimental import pallas as pl
from jax.experimental.pallas import tpu as pltpu
from jax.experimental.pallas import tpu_sc as plsc

B = 16384
L = 200
HID = 32
NLANE = 16

NC = 2
NS = 16
NW = NC * NS
SPW = B // NW

G = 8
NG = SPW // G
IDS_PER_G = G * L
CHUNK = 80
NCHUNK = IDS_PER_G // CHUNK


def _sc_embed_body(table_hbm, ids_hbm, out_hbm, idx_v, rows_v, out_v, sem):
    wid = lax.axis_index("s") * NC + lax.axis_index("c")
    inv_l = jnp.float32(1.0 / L)

    def group_body(g, carry):
        ids_base = wid * (SPW * L) + g * IDS_PER_G
        pltpu.sync_copy(ids_hbm.at[pl.ds(ids_base, IDS_PER_G)], idx_v)

        copies = []
        for j in range(NCHUNK):
            copies.append(
                pltpu.async_copy(
                    table_hbm.at[idx_v.at[pl.ds(j * CHUNK, CHUNK)]],
                    rows_v.at[pl.ds(j * CHUNK, CHUNK)],
                    sem,
                )
            )
        for c in copies:
            c.wait()

        def sample_body(s, carry2):
            rb = s * L

            def add2(l, acc):
                a0, a1, b0, b1 = acc
                r = rb + 2 * l
                a0 = a0 + rows_v[r, pl.ds(0, NLANE)]
                a1 = a1 + rows_v[r, pl.ds(NLANE, NLANE)]
                b0 = b0 + rows_v[r + 1, pl.ds(0, NLANE)]
                b1 = b1 + rows_v[r + 1, pl.ds(NLANE, NLANE)]
                return a0, a1, b0, b1

            z = jnp.zeros((NLANE,), jnp.float32)
            a0, a1, b0, b1 = lax.fori_loop(
                0, L // 2, add2, (z, z, z, z), unroll=4
            )
            out_v[s, pl.ds(0, NLANE)] = (a0 + b0) * inv_l
            out_v[s, pl.ds(NLANE, NLANE)] = (a1 + b1) * inv_l
            return carry2

        lax.fori_loop(0, G, sample_body, 0)
        pltpu.sync_copy(out_v, out_hbm.at[pl.ds(wid * SPW + g * G, G)])
        return carry

    lax.fori_loop(0, NG, group_body, 0)


@functools.cache
def _sc_embed():
    return functools.partial(
        pl.kernel,
        mesh=plsc.VectorSubcoreMesh(core_axis_name="c", subcore_axis_name="s"),
        out_type=jax.ShapeDtypeStruct((B, HID), jnp.float32),
        scratch_types=[
            pltpu.VMEM((IDS_PER_G,), jnp.int32),
            pltpu.VMEM((IDS_PER_G, HID), jnp.float32),
            pltpu.VMEM((G, HID), jnp.float32),
            pltpu.SemaphoreType.DMA,
        ],
        compiler_params=pltpu.CompilerParams(use_tc_tiling_on_sc=False),
    )(_sc_embed_body)


def _head_body(emb_ref, w_ref, b_ref, lab_ref, logits_ref, loss_ref):
    emb = emb_ref[...]
    w = w_ref[...]
    l0 = jnp.sum(emb * w[0:1, :], axis=1, keepdims=True) + b_ref[0]
    l1 = jnp.sum(emb * w[1:2, :], axis=1, keepdims=True) + b_ref[1]
    logits_ref[...] = jnp.concatenate([l0, l1], axis=1)
    m = jnp.maximum(l0, l1)
    lse = m + jnp.log(jnp.exp(l0 - m) + jnp.exp(l1 - m))
    lab = lab_ref[...]
    lp = jnp.where(lab == 0, l0, l1) - lse
    loss_ref[0, 0] = -jnp.mean(lp)


_head = pl.pallas_call(
    _head_body,
    out_shape=(
        jax.ShapeDtypeStruct((B, 2), jnp.float32),
        jax.ShapeDtypeStruct((1, 1), jnp.float32),
    ),
    in_specs=[
        pl.BlockSpec(memory_space=pltpu.VMEM),
        pl.BlockSpec(memory_space=pltpu.VMEM),
        pl.BlockSpec(memory_space=pltpu.SMEM),
        pl.BlockSpec(memory_space=pltpu.VMEM),
    ],
    out_specs=(
        pl.BlockSpec(memory_space=pltpu.VMEM),
        pl.BlockSpec(memory_space=pltpu.SMEM),
    ),
)


def kernel(input_ids, labels, emb_table, W, b):
    ids_flat = input_ids.reshape(-1)
    emb = _sc_embed()(emb_table, ids_flat)
    logits, loss = _head(emb, W, b, labels.reshape(B, 1))
    return loss[0, 0], logits


# speedup vs baseline: 22.7775x; 1.7164x over previous
import functools

import jax
import jax.numpy as jnp
from jax import lax
from jax.experimental import pallas as pl
from jax.experimental.pallas import tpu as pltpu
from jax.experimental.pallas import tpu_sc as plsc

B = 16384
L = 200
VOCAB = 1000000
HID = 32
NLANE = 16

NC = 2
NS = 16
NW = NC * NS
SPW = B // NW

G = 8
NG = SPW // G
IDS_PER_G = G * L
CHUNK = 80
NCHUNK = IDS_PER_G // CHUNK

TC_C = 8192
TC_R = TC_C // 4
TC_NMAIN = 122
MID_OFF = TC_NMAIN * TC_C
MID_C = 512
MID_R = MID_C // 4
TAIL_OFF = MID_OFF + MID_C
TAIL_C = VOCAB - TAIL_OFF
TAIL_R = TAIL_C // 4
OUT_ROWS = VOCAB * HID // 128


def _transpose_body(tT_hbm, tail_ref, out_hbm, x_buf, y_buf, in_sem, out_sem):
    g = pl.program_id(0)
    slot = lax.rem(g, 2)
    nslot = lax.rem(g + 1, 2)

    def in_copy(gg, s):
        return pltpu.make_async_copy(
            tT_hbm.at[:, pl.ds(gg * TC_C, TC_C)], x_buf.at[s], in_sem.at[s]
        )

    def out_copy(row0, s):
        return pltpu.make_async_copy(
            y_buf.at[s], out_hbm.at[pl.ds(row0, TC_R), :], out_sem.at[s]
        )

    @pl.when(g == 0)
    def _():
        in_copy(0, 0).start()

    @pl.when(g + 1 < TC_NMAIN)
    def _():
        in_copy(g + 1, nslot).start()

    @pl.when(g >= 2)
    def _():
        out_copy(0, slot).wait()

    @pl.when(g < TC_NMAIN)
    def _():
        in_copy(g, slot).wait()
        for k in range(8):
            xs = x_buf[slot, :, pl.ds(1024 * k, 1024)].T
            a = k // 2
            r0 = 1024 * (k % 2)
            y_buf[slot, pl.ds(r0, 1024), 32 * a:32 * (a + 1)] = xs
        out_copy(g * TC_R, slot).start()

    @pl.when(g == TC_NMAIN)
    def _():
        mid_in = pltpu.make_async_copy(
            tT_hbm.at[:, pl.ds(MID_OFF, MID_C)],
            x_buf.at[slot, :, pl.ds(0, MID_C)],
            in_sem.at[slot],
        )
        mid_in.start()
        mid_in.wait()
        xm = x_buf[slot, :, pl.ds(0, MID_C)].T
        for a in range(4):
            y_buf[slot, pl.ds(0, MID_R), 32 * a:32 * (a + 1)] = (
                xm[a * MID_R:(a + 1) * MID_R, :]
            )
        xtl = tail_ref[...].T
        for a in range(4):
            y_buf[slot, pl.ds(MID_R, TAIL_R), 32 * a:32 * (a + 1)] = (
                xtl[a * TAIL_R:(a + 1) * TAIL_R, :]
            )
        last = pltpu.make_async_copy(
            y_buf.at[slot, pl.ds(0, MID_R + TAIL_R), :],
            out_hbm.at[pl.ds(MID_OFF // 4, MID_R + TAIL_R), :],
            out_sem.at[slot],
        )
        last.start()
        last.wait()
        out_copy(0, nslot).wait()


_table_to_linear = pl.pallas_call(
    _transpose_body,
    grid=(TC_NMAIN + 1,),
    in_specs=[
        pl.BlockSpec(memory_space=pl.ANY),
        pl.BlockSpec(memory_space=pltpu.VMEM),
    ],
    out_specs=pl.BlockSpec(memory_space=pl.ANY),
    out_shape=jax.ShapeDtypeStruct((OUT_ROWS, 128), jnp.float32),
    scratch_shapes=[
        pltpu.VMEM((2, HID, TC_C), jnp.float32),
        pltpu.VMEM((2, TC_R, 128), jnp.float32),
        pltpu.SemaphoreType.DMA((2,)),
        pltpu.SemaphoreType.DMA((2,)),
    ],
)


def _permute_ids(ids):
    main = (
        ((ids >> 13) << 13)
        + ((ids & 2047) << 2)
        + ((ids & 8191) >> 11)
    )
    vm = ids - MID_OFF
    mid = MID_OFF + ((vm & 127) << 2) + (vm >> 7)
    vt = ids - TAIL_OFF
    tail = TAIL_OFF + ((vt & 15) << 2) + (vt >> 4)
    return jnp.where(
        ids < MID_OFF, main, jnp.where(ids < TAIL_OFF, mid, tail)
    )


def _sc_embed_body(table_hbm, ids_hbm, out_hbm, idx_v, rows_v, out_v, sem0, sem1):
    wid = lax.axis_index("s") * NC + lax.axis_index("c")
    inv_l = jnp.float32(1.0 / L)

    def start_group(g, slot, sem):
        ids_base = wid * (SPW * L) + g * IDS_PER_G
        pltpu.sync_copy(ids_hbm.at[pl.ds(ids_base, IDS_PER_G)], idx_v.at[slot])
        for j in range(NCHUNK):
            pltpu.async_copy(
                table_hbm.at[idx_v.at[slot, pl.ds(j * CHUNK, CHUNK)]],
                rows_v.at[slot, pl.ds(j * CHUNK, CHUNK)],
                sem,
            )

    def drain_group(slot, sem):
        for j in range(NCHUNK):
            pltpu.make_async_copy(
                table_hbm.at[idx_v.at[slot, pl.ds(j * CHUNK, CHUNK)]],
                rows_v.at[slot, pl.ds(j * CHUNK, CHUNK)],
                sem,
            ).wait()

    start_group(0, 0, sem0)

    def group_body(g, carry):
        slot = lax.rem(g, 2)
        nslot = lax.rem(g + 1, 2)

        @pl.when(g + 1 < NG)
        def _():
            @pl.when(nslot == 1)
            def _():
                start_group(g + 1, 1, sem1)

            @pl.when(nslot == 0)
            def _():
                start_group(g + 1, 0, sem0)

        @pl.when(slot == 0)
        def _():
            drain_group(0, sem0)

        @pl.when(slot == 1)
        def _():
            drain_group(1, sem1)

        def sample_body(s, carry2):
            rb = s * L

            def add2(l, acc):
                a0, a1, b0, b1 = acc
                r = rb + 2 * l
                a0 = a0 + rows_v[slot, r, pl.ds(0, NLANE)]
                a1 = a1 + rows_v[slot, r, pl.ds(NLANE, NLANE)]
                b0 = b0 + rows_v[slot, r + 1, pl.ds(0, NLANE)]
                b1 = b1 + rows_v[slot, r + 1, pl.ds(NLANE, NLANE)]
                return a0, a1, b0, b1

            z = jnp.zeros((NLANE,), jnp.float32)
            a0, a1, b0, b1 = lax.fori_loop(
                0, L // 2, add2, (z, z, z, z), unroll=4
            )
            out_v[s, pl.ds(0, NLANE)] = (a0 + b0) * inv_l
            out_v[s, pl.ds(NLANE, NLANE)] = (a1 + b1) * inv_l
            return carry2

        lax.fori_loop(0, G, sample_body, 0)
        pltpu.sync_copy(out_v, out_hbm.at[pl.ds(wid * SPW + g * G, G)])
        return carry

    lax.fori_loop(0, NG, group_body, 0)


@functools.cache
def _sc_embed():
    return functools.partial(
        pl.kernel,
        mesh=plsc.VectorSubcoreMesh(core_axis_name="c", subcore_axis_name="s"),
        out_type=jax.ShapeDtypeStruct((B, HID), jnp.float32),
        scratch_types=[
            pltpu.VMEM((2, IDS_PER_G), jnp.int32),
            pltpu.VMEM((2, IDS_PER_G, HID), jnp.float32),
            pltpu.VMEM((G, HID), jnp.float32),
            pltpu.SemaphoreType.DMA,
            pltpu.SemaphoreType.DMA,
        ],
        compiler_params=pltpu.CompilerParams(use_tc_tiling_on_sc=False),
    )(_sc_embed_body)


def _head_body(emb_ref, w_ref, b_ref, lab_ref, logits_ref, loss_ref):
    emb = emb_ref[...]
    w = w_ref[...]
    l0 = jnp.sum(emb * w[0:1, :], axis=1, keepdims=True) + b_ref[0]
    l1 = jnp.sum(emb * w[1:2, :], axis=1, keepdims=True) + b_ref[1]
    logits_ref[...] = jnp.concatenate([l0, l1], axis=1)
    m = jnp.maximum(l0, l1)
    lse = m + jnp.log(jnp.exp(l0 - m) + jnp.exp(l1 - m))
    lab = lab_ref[...]
    lp = jnp.where(lab == 0, l0, l1) - lse
    loss_ref[0, 0] = -jnp.mean(lp)


_head = pl.pallas_call(
    _head_body,
    out_shape=(
        jax.ShapeDtypeStruct((B, 2), jnp.float32),
        jax.ShapeDtypeStruct((1, 1), jnp.float32),
    ),
    in_specs=[
        pl.BlockSpec(memory_space=pltpu.VMEM),
        pl.BlockSpec(memory_space=pltpu.VMEM),
        pl.BlockSpec(memory_space=pltpu.SMEM),
        pl.BlockSpec(memory_space=pltpu.VMEM),
    ],
    out_specs=(
        pl.BlockSpec(memory_space=pltpu.VMEM),
        pl.BlockSpec(memory_space=pltpu.SMEM),
    ),
)


def kernel(input_ids, labels, emb_table, W, b):
    tail = lax.slice(emb_table, (TAIL_OFF, 0), (VOCAB, HID)).T
    table_lin = _table_to_linear(emb_table.T, tail)
    table_sc = table_lin.reshape(VOCAB * HID).reshape(VOCAB, HID)
    ids_flat = _permute_ids(input_ids).reshape(-1)
    emb = _sc_embed()(table_sc, ids_flat)
    logits, loss = _head(emb, W, b, labels.reshape(B, 1))
    return loss[0, 0], logits
